# Initial kernel scaffold; baseline (speedup 1.0000x reference)
import jax
import jax.numpy as jnp
from jax.experimental import pallas as pl


def kernel(pc0_pos, pc0_feat, pc0_batch, pc1_pos, pc1_feat, pc1_batch, k, kp, w_d0, g_d0, b_d0, w_d1, g_d1, b_d1, w_d2, g_d2, b_d2, w_d3, g_d3, b_d3, w_u0, g_u0, b_u0, w_u1, g_u1, b_u1, w_u2, g_u2, b_u2, fc_w1, fc_g, fc_b, fc_w2):
    raise NotImplementedError("write your pallas kernel here")


# speedup vs baseline: 3.6067x; 3.6067x over previous
import functools

import jax
import jax.numpy as jnp
from jax import lax
from jax.experimental import pallas as pl
from jax.experimental.pallas import tpu as pltpu
from jax.experimental.pallas import tpu_sc as plsc

N = 8192
K = 16
KP_N = 25
SIGMA = 0.1
NC, NS = 2, 16
NW = NC * NS
F32 = jnp.float32
I32 = jnp.int32


def _sc_gather_body(idx_hbm, table_hbm, out_hbm, idx_v, r0, r1, sg0, sg1,
                    sw0, sw1, *, per_w, chunk, nch):
    wid = lax.axis_index("s") * NC + lax.axis_index("c")
    base = wid * per_w
    pltpu.sync_copy(idx_hbm.at[pl.ds(base, per_w)], idx_v)
    bufs = ((r0, sg0, sw0), (r1, sg1, sw1))

    def step(p, carry):
        for b in range(2):
            j = p * 2 + b
            r, sg, sw = bufs[b]

            @pl.when(p > 0)
            def _():
                pltpu.make_async_copy(
                    r, out_hbm.at[pl.ds(base + (j - 2) * chunk, chunk)], sw
                ).wait()

            pltpu.async_copy(table_hbm.at[idx_v.at[pl.ds(j * chunk, chunk)]],
                             r, sg)
        for b in range(2):
            j = p * 2 + b
            r, sg, sw = bufs[b]
            pltpu.make_async_copy(
                table_hbm.at[idx_v.at[pl.ds(j * chunk, chunk)]], r, sg).wait()
            pltpu.async_copy(r, out_hbm.at[pl.ds(base + j * chunk, chunk)], sw)
        return carry

    lax.fori_loop(0, nch // 2, step, 0)
    for b in range(2):
        r, sg, sw = bufs[b]
        j = nch - 2 + b
        pltpu.make_async_copy(
            r, out_hbm.at[pl.ds(base + j * chunk, chunk)], sw).wait()


@functools.partial(jax.jit, static_argnums=(2,))
def _sc_gather(table, idx, cols):
    m = idx.shape[0]
    per_w = m // NW
    chunk = min(128, per_w)
    nch = per_w // chunk
    assert nch % 2 == 0 or nch == 1, (m, per_w, chunk, nch)
    if nch == 1:
        chunk = per_w // 2
        nch = 2
    mesh = plsc.VectorSubcoreMesh(core_axis_name="c", subcore_axis_name="s",
                                  num_cores=NC, num_subcores=NS)
    body = functools.partial(_sc_gather_body, per_w=per_w, chunk=chunk,
                             nch=nch)
    return pl.kernel(
        body,
        out_type=jax.ShapeDtypeStruct((m, cols), F32),
        mesh=mesh,
        compiler_params=pltpu.CompilerParams(use_tc_tiling_on_sc=False),
        scratch_types=[
            pltpu.VMEM((per_w,), I32),
            pltpu.VMEM((chunk, cols), F32),
            pltpu.VMEM((chunk, cols), F32),
            pltpu.SemaphoreType.DMA,
            pltpu.SemaphoreType.DMA,
            pltpu.SemaphoreType.DMA,
            pltpu.SemaphoreType.DMA,
        ],
    )(idx, table)


def _knn_body(qp_ref, st_ref, out_ref, *, nk, t):
    q = qp_ref[...]
    st = st_ref[...]
    d = (jnp.sum(q * q, axis=1, keepdims=True)
         - 2.0 * jnp.dot(q[:, 0:3], st[0:3, :], preferred_element_type=F32)
         + jnp.sum(st * st, axis=0, keepdims=True))
    cols = lax.broadcasted_iota(I32, (t, N), 1)
    for kk in range(nk):
        mval = jnp.min(d, axis=1, keepdims=True)
        loc = jnp.where(d == mval, cols, jnp.int32(N))
        idx = jnp.min(loc, axis=1)
        out_ref[kk, :] = idx
        if kk + 1 < nk:
            d = jnp.where(cols == idx[:, None], jnp.float32(jnp.inf), d)


def _knn(qpad, st, nk, t=128):
    grid = qpad.shape[0] // t
    return pl.pallas_call(
        functools.partial(_knn_body, nk=nk, t=t),
        grid=(grid,),
        in_specs=[
            pl.BlockSpec((t, 8), lambda i: (i, 0)),
            pl.BlockSpec((8, N), lambda i: (0, 0)),
        ],
        out_specs=pl.BlockSpec((nk, t), lambda i: (0, i)),
        out_shape=jax.ShapeDtypeStruct((nk, qpad.shape[0]), I32),
    )(qpad, st)


def _wk_body(g0_ref, pp_ref, kpt_ref, w_ref, *, t):
    kpt = kpt_ref[...]
    kp2 = jnp.sum(kpt * kpt, axis=0, keepdims=True)
    p = pp_ref[...]
    for kk in range(K):
        gk = g0_ref[kk]
        rel = gk[:, 0:8] - p
        rr = jnp.sum(rel[:, 0:3] * rel[:, 0:3], axis=1, keepdims=True)
        d2 = rr - 2.0 * jnp.dot(rel, kpt, preferred_element_type=F32) + kp2
        d = jnp.sqrt(jnp.maximum(d2, 0.0) + 1e-12)
        w_ref[kk] = jnp.maximum(0.0, 1.0 - d / SIGMA)


def _wkern(g0, ppad, kpt, t=256):
    grid = N // t
    return pl.pallas_call(
        functools.partial(_wk_body, t=t),
        grid=(grid,),
        in_specs=[
            pl.BlockSpec((K, t, 16), lambda i: (0, i, 0)),
            pl.BlockSpec((t, 8), lambda i: (i, 0)),
            pl.BlockSpec((8, KP_N), lambda i: (0, 0)),
        ],
        out_specs=pl.BlockSpec((K, t, KP_N), lambda i: (0, i, 0)),
        out_shape=jax.ShapeDtypeStruct((K, N, KP_N), F32),
    )(g0, ppad, kpt)


def _conv_body(nf_ref, w_ref, wm_ref, y_ref, *, c, d, t):
    bf = jnp.bfloat16
    nfb = [nf_ref[kk].astype(bf).astype(F32) for kk in range(K)]
    wb = w_ref[...].astype(bf).astype(F32)
    acc = jnp.zeros((t, d), F32)
    for m in range(KP_N):
        fkm = wb[0, :, m:m + 1] * nfb[0]
        for kk in range(1, K):
            fkm = fkm + wb[kk, :, m:m + 1] * nfb[kk]
        acc = acc + jnp.dot(fkm.astype(bf), wm_ref[m].astype(bf),
                            preferred_element_type=F32)
    y_ref[...] = acc


def _conv(nf, w, wm, t=128):
    c, d = wm.shape[1], wm.shape[2]
    grid = N // t
    return pl.pallas_call(
        functools.partial(_conv_body, c=c, d=d, t=t),
        grid=(grid,),
        in_specs=[
            pl.BlockSpec((K, t, c), lambda i: (0, i, 0)),
            pl.BlockSpec((K, t, KP_N), lambda i: (0, i, 0)),
            pl.BlockSpec((KP_N, c, d), lambda i: (0, 0, 0)),
        ],
        out_specs=pl.BlockSpec((t, d), lambda i: (i, 0)),
        out_shape=jax.ShapeDtypeStruct((N, d), F32),
    )(nf, w, wm)


def _bn_body(y_ref, g_ref, b_ref, o_ref):
    y = y_ref[...]
    m = jnp.mean(y, axis=0, keepdims=True)
    yc = y - m
    v = jnp.mean(yc * yc, axis=0, keepdims=True)
    z = yc / jnp.sqrt(v + 1e-5) * g_ref[...] + b_ref[...]
    o_ref[...] = jnp.where(z >= 0.0, z, 0.2 * z)


def _bn_sub_body(y_ref, g_ref, b_ref, s_ref, o_ref, d_ref):
    y = y_ref[...]
    m = jnp.mean(y, axis=0, keepdims=True)
    yc = y - m
    v = jnp.mean(yc * yc, axis=0, keepdims=True)
    z = yc / jnp.sqrt(v + 1e-5) * g_ref[...] + b_ref[...]
    o = jnp.where(z >= 0.0, z, 0.2 * z)
    o_ref[...] = o
    d_ref[...] = o - s_ref[...]


def _bn(y, g, b):
    d = y.shape[1]
    return pl.pallas_call(
        _bn_body,
        out_shape=jax.ShapeDtypeStruct((N, d), F32),
    )(y, g.reshape(1, d), b.reshape(1, d))


def _bn_sub(y, g, b, s):
    d = y.shape[1]
    return pl.pallas_call(
        _bn_sub_body,
        out_shape=(jax.ShapeDtypeStruct((N, d), F32),
                   jax.ShapeDtypeStruct((N, d), F32)),
    )(y, g.reshape(1, d), b.reshape(1, d), s)


def _fc_body(x_ref, w1_ref, g_ref, b_ref, w2_ref, o_ref):
    bf = jnp.bfloat16
    h = jnp.dot(x_ref[...].astype(bf), w1_ref[...].astype(bf),
                preferred_element_type=F32)
    m = jnp.mean(h, axis=0, keepdims=True)
    hc = h - m
    v = jnp.mean(hc * hc, axis=0, keepdims=True)
    z = hc / jnp.sqrt(v + 1e-5) * g_ref[...] + b_ref[...]
    z = jnp.where(z >= 0.0, z, 0.2 * z)
    o_ref[...] = jnp.dot(z.astype(bf), w2_ref[...].astype(bf),
                         preferred_element_type=F32)


def _fc(x, w1, g, b, w2):
    return pl.pallas_call(
        _fc_body,
        out_shape=jax.ShapeDtypeStruct((N, w2.shape[1]), F32),
    )(x, w1, g.reshape(1, -1), b.reshape(1, -1), w2)


def kernel(pc0_pos, pc0_feat, pc0_batch, pc1_pos, pc1_feat, pc1_batch, k, kp,
           w_d0, g_d0, b_d0, w_d1, g_d1, b_d1, w_d2, g_d2, b_d2, w_d3, g_d3,
           b_d3, w_u0, g_u0, b_u0, w_u1, g_u1, b_u1, w_u2, g_u2, b_u2, fc_w1,
           fc_g, fc_b, fc_w2):
    koff = jnp.asarray(k, I32) - K

    p0 = jnp.pad(pc0_pos, ((0, 0), (0, 5)))
    p1 = jnp.pad(pc1_pos, ((0, 0), (0, 5)))
    p0t = p0.T
    p1t = p1.T
    kpt = jnp.pad(kp, ((0, 0), (0, 5))).T

    n1t = _knn(p0, p0t, K) + koff
    n2t = _knn(p1, p1t, K) + koff
    cross = _knn(p1, p0t, 1).reshape(N)
    idx1 = n1t.reshape(-1)
    idx2 = n2t.reshape(-1)

    t1 = jnp.pad(jnp.concatenate([pc0_pos, pc0_feat], axis=1),
                 ((0, 0), (0, 10)))
    t2 = jnp.pad(jnp.concatenate([pc1_pos, pc1_feat], axis=1),
                 ((0, 0), (0, 10)))
    g01 = _sc_gather(t1, idx1, 16).reshape(K, N, 16)
    g02 = _sc_gather(t2, idx2, 16).reshape(K, N, 16)
    w1 = _wkern(g01, p0, kpt)
    w2 = _wkern(g02, p1, kpt)

    wd = [jnp.pad(w_d0, ((0, 0), (0, 10), (0, 0))), w_d1, w_d2, w_d3]
    gd = [g_d0, g_d1, g_d2, g_d3]
    bd = [b_d0, b_d1, b_d2, b_d3]
    wu = [w_u0, w_u1, w_u2]
    gu = [g_u0, g_u1, g_u2]
    bu = [b_u0, b_u1, b_u2]

    nf1, nf2 = g01, g02
    stack = []
    x = None
    for i in range(4):
        dch = wd[i].shape[2]
        y1 = _conv(nf1, w1, wd[i])
        x1f = _bn(y1, gd[i], bd[i])
        x1c = _sc_gather(x1f, cross, dch)
        y2 = _conv(nf2, w2, wd[i])
        x2f, diff = _bn_sub(y2, gd[i], bd[i], x1c)
        if i < 3:
            stack.append(diff)
            nf1 = _sc_gather(x1f, idx1, dch).reshape(K, N, dch)
            nf2 = _sc_gather(x2f, idx2, dch).reshape(K, N, dch)
        else:
            x = diff

    for i in range(3):
        tab = jnp.concatenate([x, stack.pop()], axis=1)
        cch = tab.shape[1]
        nfu = _sc_gather(tab, idx2, cch).reshape(K, N, cch)
        yu = _conv(nfu, w2, wu[i])
        x = _bn(yu, gu[i], bu[i])

    return _fc(x, fc_w1, fc_g, fc_b, fc_w2)


# speedup vs baseline: 3.7412x; 1.0373x over previous
import functools

import jax
import jax.numpy as jnp
from jax import lax
from jax.experimental import pallas as pl
from jax.experimental.pallas import tpu as pltpu
from jax.experimental.pallas import tpu_sc as plsc

N = 8192
K = 16
KP_N = 25
SIGMA = 0.1
NC, NS = 2, 16
NW = NC * NS
F32 = jnp.float32
I32 = jnp.int32


def _sc_gather_body(idx_hbm, table_hbm, out_hbm, idx_v, r0, r1, sg0, sg1,
                    sw0, sw1, *, per_w, chunk, nch):
    wid = lax.axis_index("s") * NC + lax.axis_index("c")
    base = wid * per_w
    pltpu.sync_copy(idx_hbm.at[pl.ds(base, per_w)], idx_v)
    bufs = ((r0, sg0, sw0), (r1, sg1, sw1))

    def step(p, carry):
        for b in range(2):
            j = p * 2 + b
            r, sg, sw = bufs[b]

            @pl.when(p > 0)
            def _():
                pltpu.make_async_copy(
                    r, out_hbm.at[pl.ds(base + (j - 2) * chunk, chunk)], sw
                ).wait()

            pltpu.async_copy(table_hbm.at[idx_v.at[pl.ds(j * chunk, chunk)]],
                             r, sg)
        for b in range(2):
            j = p * 2 + b
            r, sg, sw = bufs[b]
            pltpu.make_async_copy(
                table_hbm.at[idx_v.at[pl.ds(j * chunk, chunk)]], r, sg).wait()
            pltpu.async_copy(r, out_hbm.at[pl.ds(base + j * chunk, chunk)], sw)
        return carry

    lax.fori_loop(0, nch // 2, step, 0)
    for b in range(2):
        r, sg, sw = bufs[b]
        j = nch - 2 + b
        pltpu.make_async_copy(
            r, out_hbm.at[pl.ds(base + j * chunk, chunk)], sw).wait()


@functools.partial(jax.jit, static_argnums=(2,))
def _sc_gather(table, idx, cols):
    m = idx.shape[0]
    per_w = m // NW
    chunk = min(128, per_w)
    nch = per_w // chunk
    assert nch % 2 == 0 or nch == 1, (m, per_w, chunk, nch)
    if nch == 1:
        chunk = per_w // 2
        nch = 2
    mesh = plsc.VectorSubcoreMesh(core_axis_name="c", subcore_axis_name="s",
                                  num_cores=NC, num_subcores=NS)
    body = functools.partial(_sc_gather_body, per_w=per_w, chunk=chunk,
                             nch=nch)
    return pl.kernel(
        body,
        out_type=jax.ShapeDtypeStruct((m, cols), F32),
        mesh=mesh,
        compiler_params=pltpu.CompilerParams(use_tc_tiling_on_sc=False),
        scratch_types=[
            pltpu.VMEM((per_w,), I32),
            pltpu.VMEM((chunk, cols), F32),
            pltpu.VMEM((chunk, cols), F32),
            pltpu.SemaphoreType.DMA,
            pltpu.SemaphoreType.DMA,
            pltpu.SemaphoreType.DMA,
            pltpu.SemaphoreType.DMA,
        ],
    )(idx, table)


def _knn_body(qp_ref, st_ref, out_ref, *, nk, t):
    q = qp_ref[...]
    st = st_ref[...]
    d = (jnp.sum(q * q, axis=1, keepdims=True)
         - 2.0 * jnp.dot(q[:, 0:3], st[0:3, :], preferred_element_type=F32)
         + jnp.sum(st * st, axis=0, keepdims=True))
    cols = lax.broadcasted_iota(I32, (t, N), 1)
    for kk in range(nk):
        idx = jnp.argmin(d, axis=1).astype(I32)
        out_ref[kk, :] = idx
        if kk + 1 < nk:
            d = jnp.where(cols == idx[:, None], jnp.float32(jnp.inf), d)


def _knn(qpad, st, nk, t=128):
    grid = qpad.shape[0] // t
    return pl.pallas_call(
        functools.partial(_knn_body, nk=nk, t=t),
        grid=(grid,),
        in_specs=[
            pl.BlockSpec((t, 8), lambda i: (i, 0)),
            pl.BlockSpec((8, N), lambda i: (0, 0)),
        ],
        out_specs=pl.BlockSpec((nk, t), lambda i: (0, i)),
        out_shape=jax.ShapeDtypeStruct((nk, qpad.shape[0]), I32),
    )(qpad, st)


def _wk_body(g0_ref, pp_ref, kpt_ref, w_ref, *, t):
    kpt = kpt_ref[...]
    kp2 = jnp.sum(kpt * kpt, axis=0, keepdims=True)
    p = pp_ref[...]
    for kk in range(K):
        gk = g0_ref[kk]
        rel = gk[:, 0:8] - p
        rr = jnp.sum(rel[:, 0:3] * rel[:, 0:3], axis=1, keepdims=True)
        d2 = rr - 2.0 * jnp.dot(rel, kpt, preferred_element_type=F32) + kp2
        d = jnp.sqrt(jnp.maximum(d2, 0.0) + 1e-12)
        w_ref[kk] = jnp.maximum(0.0, 1.0 - d / SIGMA)


def _wkern(g0, ppad, kpt, t=256):
    grid = N // t
    return pl.pallas_call(
        functools.partial(_wk_body, t=t),
        grid=(grid,),
        in_specs=[
            pl.BlockSpec((K, t, 16), lambda i: (0, i, 0)),
            pl.BlockSpec((t, 8), lambda i: (i, 0)),
            pl.BlockSpec((8, KP_N), lambda i: (0, 0)),
        ],
        out_specs=pl.BlockSpec((K, t, KP_N), lambda i: (0, i, 0)),
        out_shape=jax.ShapeDtypeStruct((K, N, KP_N), F32),
    )(g0, ppad, kpt)


def _conv_body(nf_ref, w_ref, wm_ref, y_ref, *, c, d, t):
    bf = jnp.bfloat16
    nfb = [nf_ref[kk].astype(bf).astype(F32) for kk in range(K)]
    wb = w_ref[...].astype(bf).astype(F32)
    acc = jnp.zeros((t, d), F32)
    for m in range(KP_N):
        fkm = wb[0, :, m:m + 1] * nfb[0]
        for kk in range(1, K):
            fkm = fkm + wb[kk, :, m:m + 1] * nfb[kk]
        acc = acc + jnp.dot(fkm.astype(bf), wm_ref[m].astype(bf),
                            preferred_element_type=F32)
    y_ref[...] = acc


def _conv(nf, w, wm, t=128):
    c, d = wm.shape[1], wm.shape[2]
    grid = N // t
    return pl.pallas_call(
        functools.partial(_conv_body, c=c, d=d, t=t),
        grid=(grid,),
        in_specs=[
            pl.BlockSpec((K, t, c), lambda i: (0, i, 0)),
            pl.BlockSpec((K, t, KP_N), lambda i: (0, i, 0)),
            pl.BlockSpec((KP_N, c, d), lambda i: (0, 0, 0)),
        ],
        out_specs=pl.BlockSpec((t, d), lambda i: (i, 0)),
        out_shape=jax.ShapeDtypeStruct((N, d), F32),
    )(nf, w, wm)


def _bn_body(y_ref, g_ref, b_ref, o_ref):
    y = y_ref[...]
    m = jnp.mean(y, axis=0, keepdims=True)
    yc = y - m
    v = jnp.mean(yc * yc, axis=0, keepdims=True)
    z = yc / jnp.sqrt(v + 1e-5) * g_ref[...] + b_ref[...]
    o_ref[...] = jnp.where(z >= 0.0, z, 0.2 * z)


def _bn_sub_body(y_ref, g_ref, b_ref, s_ref, o_ref, d_ref):
    y = y_ref[...]
    m = jnp.mean(y, axis=0, keepdims=True)
    yc = y - m
    v = jnp.mean(yc * yc, axis=0, keepdims=True)
    z = yc / jnp.sqrt(v + 1e-5) * g_ref[...] + b_ref[...]
    o = jnp.where(z >= 0.0, z, 0.2 * z)
    o_ref[...] = o
    d_ref[...] = o - s_ref[...]


def _bn(y, g, b):
    d = y.shape[1]
    return pl.pallas_call(
        _bn_body,
        out_shape=jax.ShapeDtypeStruct((N, d), F32),
    )(y, g.reshape(1, d), b.reshape(1, d))


def _bn_sub(y, g, b, s):
    d = y.shape[1]
    return pl.pallas_call(
        _bn_sub_body,
        out_shape=(jax.ShapeDtypeStruct((N, d), F32),
                   jax.ShapeDtypeStruct((N, d), F32)),
    )(y, g.reshape(1, d), b.reshape(1, d), s)


def _fc_body(x_ref, w1_ref, g_ref, b_ref, w2_ref, o_ref):
    bf = jnp.bfloat16
    h = jnp.dot(x_ref[...].astype(bf), w1_ref[...].astype(bf),
                preferred_element_type=F32)
    m = jnp.mean(h, axis=0, keepdims=True)
    hc = h - m
    v = jnp.mean(hc * hc, axis=0, keepdims=True)
    z = hc / jnp.sqrt(v + 1e-5) * g_ref[...] + b_ref[...]
    z = jnp.where(z >= 0.0, z, 0.2 * z)
    o_ref[...] = jnp.dot(z.astype(bf), w2_ref[...].astype(bf),
                         preferred_element_type=F32)


def _fc(x, w1, g, b, w2):
    return pl.pallas_call(
        _fc_body,
        out_shape=jax.ShapeDtypeStruct((N, w2.shape[1]), F32),
    )(x, w1, g.reshape(1, -1), b.reshape(1, -1), w2)


def kernel(pc0_pos, pc0_feat, pc0_batch, pc1_pos, pc1_feat, pc1_batch, k, kp,
           w_d0, g_d0, b_d0, w_d1, g_d1, b_d1, w_d2, g_d2, b_d2, w_d3, g_d3,
           b_d3, w_u0, g_u0, b_u0, w_u1, g_u1, b_u1, w_u2, g_u2, b_u2, fc_w1,
           fc_g, fc_b, fc_w2):
    koff = jnp.asarray(k, I32) - K

    p0 = jnp.pad(pc0_pos, ((0, 0), (0, 5)))
    p1 = jnp.pad(pc1_pos, ((0, 0), (0, 5)))
    p0t = p0.T
    p1t = p1.T
    kpt = jnp.pad(kp, ((0, 0), (0, 5))).T

    n1t = _knn(p0, p0t, K) + koff
    n2t = _knn(p1, p1t, K) + koff
    cross = _knn(p1, p0t, 1).reshape(N)
    idx1 = n1t.reshape(-1)
    idx2 = n2t.reshape(-1)

    t1 = jnp.pad(jnp.concatenate([pc0_pos, pc0_feat], axis=1),
                 ((0, 0), (0, 10)))
    t2 = jnp.pad(jnp.concatenate([pc1_pos, pc1_feat], axis=1),
                 ((0, 0), (0, 10)))
    g01 = _sc_gather(t1, idx1, 16).reshape(K, N, 16)
    g02 = _sc_gather(t2, idx2, 16).reshape(K, N, 16)
    w1 = _wkern(g01, p0, kpt)
    w2 = _wkern(g02, p1, kpt)

    wd = [jnp.pad(w_d0, ((0, 0), (0, 10), (0, 0))), w_d1, w_d2, w_d3]
    gd = [g_d0, g_d1, g_d2, g_d3]
    bd = [b_d0, b_d1, b_d2, b_d3]
    wu = [w_u0, w_u1, w_u2]
    gu = [g_u0, g_u1, g_u2]
    bu = [b_u0, b_u1, b_u2]

    nf1, nf2 = g01, g02
    stack = []
    x = None
    for i in range(4):
        dch = wd[i].shape[2]
        y1 = _conv(nf1, w1, wd[i])
        x1f = _bn(y1, gd[i], bd[i])
        x1c = _sc_gather(x1f, cross, dch)
        y2 = _conv(nf2, w2, wd[i])
        x2f, diff = _bn_sub(y2, gd[i], bd[i], x1c)
        if i < 3:
            stack.append(diff)
            nf1 = _sc_gather(x1f, idx1, dch).reshape(K, N, dch)
            nf2 = _sc_gather(x2f, idx2, dch).reshape(K, N, dch)
        else:
            x = diff

    for i in range(3):
        tab = jnp.concatenate([x, stack.pop()], axis=1)
        cch = tab.shape[1]
        nfu = _sc_gather(tab, idx2, cch).reshape(K, N, cch)
        yu = _conv(nfu, w2, wu[i])
        x = _bn(yu, gu[i], bu[i])

    return _fc(x, fc_w1, fc_g, fc_b, fc_w2)
